# baseline (speedup 1.0000x reference)
import functools

import jax
import jax.numpy as jnp
from jax import lax
from jax.experimental import pallas as pl
from jax.experimental.pallas import tpu as pltpu
from jax.experimental.pallas import tpu_sc as plsc

N_NODES = 99996
N_EDGES = 3199872
HID = 32

NPAD = 100608
EPAD = 3211264
NW = 32
CH = 1024
CROWS = CH // 128
EROWS = EPAD // 128
ROWS_TILE = EROWS // NW
N_CHUNKS = ROWS_TILE // CROWS
STRIPE = NPAD // 16
BLK = 768
GRID = NPAD // BLK
OUT_BLK = BLK // 6
TW = 16

_mesh = plsc.VectorSubcoreMesh(core_axis_name="c", subcore_axis_name="s")


@functools.partial(
    pl.kernel,
    mesh=_mesh,
    out_type=jax.ShapeDtypeStruct((NW, NPAD), jnp.float32),
    scratch_types=[
        pltpu.VMEM((NPAD,), jnp.float32),
        pltpu.VMEM((16, 128), jnp.int32),
    ],
    compiler_params=pltpu.CompilerParams(
        needs_layout_passes=False, use_tc_tiling_on_sc=False),
)
def _sc_count(dst_hbm, out_hbm, cnt_v, idx_v):
    c = lax.axis_index("c")
    s = lax.axis_index("s")
    w = s * 2 + c

    def zero_body(j, carry):
        cnt_v[pl.ds(j * 16, 16)] = jnp.zeros((16,), jnp.float32)
        return carry

    lax.fori_loop(0, NPAD // 16, zero_body, 0)

    ones = jnp.ones((16,), jnp.float32)

    def chunk_body(g, carry):
        rowbase = w * ROWS_TILE + g * 16
        pltpu.sync_copy(dst_hbm.at[pl.ds(rowbase, 16)], idx_v)

        def inner(t, carry2):
            j = t // 8
            l = t - j * 8
            idx = idx_v[j, pl.ds(l * 16, 16)]
            plsc.addupdate_scatter(cnt_v, [idx], ones)
            return carry2

        lax.fori_loop(0, 128, inner, 0)
        return carry

    lax.fori_loop(0, ROWS_TILE // 16, chunk_body, 0)
    pltpu.sync_copy(cnt_v, out_hbm.at[w])


@functools.partial(
    pl.kernel,
    mesh=_mesh,
    out_type=jax.ShapeDtypeStruct((2, NPAD, TW), jnp.bfloat16),
    scratch_types=[
        pltpu.VMEM((CROWS, 128), jnp.int32),
        pltpu.VMEM((CROWS, 128), jnp.int32),
        pltpu.VMEM((CROWS, 128), jnp.int32),
        pltpu.VMEM((CROWS, 128), jnp.int32),
        pltpu.VMEM((CH, TW), jnp.bfloat16),
        pltpu.VMEM((CH, TW), jnp.bfloat16),
        pltpu.VMEM_SHARED((NPAD, TW), jnp.bfloat16),
        pltpu.SemaphoreType.DMA,
        pltpu.SemaphoreType.DMA,
    ],
    compiler_params=pltpu.CompilerParams(
        needs_layout_passes=False, use_tc_tiling_on_sc=False),
)
def _sc_aggregate(src_hbm, dst_hbm, tab_hbm, zeros_hbm, out_hbm,
                  si_a, di_a, si_b, di_b, rows_a, rows_b,
                  acc_sh, gsem, ssem):
    c = lax.axis_index("c")
    s = lax.axis_index("s")
    w = s * 2 + c
    row0 = s * STRIPE

    pltpu.sync_copy(zeros_hbm.at[pl.ds(row0, STRIPE)],
                    acc_sh.at[pl.ds(row0, STRIPE)])
    plsc.subcore_barrier()

    def gather_chunk(chunk, si_v, di_v, rows_v):
        rowbase = w * ROWS_TILE + chunk * CROWS
        pltpu.sync_copy(src_hbm.at[pl.ds(rowbase, CROWS)], si_v)
        pltpu.sync_copy(dst_hbm.at[pl.ds(rowbase, CROWS)], di_v)
        gathers = []
        for j in range(CROWS):
            gathers.append(pltpu.async_copy(
                tab_hbm.at[si_v.at[j]],
                rows_v.at[pl.ds(j * 128, 128)],
                gsem,
            ))
        for d in gathers:
            d.wait()

    def fire_scatters(di_v, rows_v):
        for j in range(CROWS):
            pltpu.async_copy(
                rows_v.at[pl.ds(j * 128, 128)],
                acc_sh.at[di_v.at[j]],
                ssem,
                add=True,
            )

    def drain_scatters(di_v, rows_v):
        for j in range(CROWS):
            pltpu.make_async_copy(
                rows_v.at[pl.ds(j * 128, 128)],
                acc_sh.at[di_v.at[j]],
                ssem,
            ).wait()

    gather_chunk(0, si_a, di_a, rows_a)
    fire_scatters(di_a, rows_a)
    gather_chunk(1, si_b, di_b, rows_b)
    fire_scatters(di_b, rows_b)

    def body(g, carry):
        drain_scatters(di_a, rows_a)
        gather_chunk(2 * g, si_a, di_a, rows_a)
        fire_scatters(di_a, rows_a)
        drain_scatters(di_b, rows_b)
        gather_chunk(2 * g + 1, si_b, di_b, rows_b)
        fire_scatters(di_b, rows_b)
        return carry

    lax.fori_loop(1, N_CHUNKS // 2, body, 0)
    drain_scatters(di_a, rows_a)
    drain_scatters(di_b, rows_b)

    plsc.subcore_barrier()
    pltpu.sync_copy(acc_sh.at[pl.ds(row0, STRIPE)],
                    out_hbm.at[c, pl.ds(row0, STRIPE)])


def _tc_prep_body(state_ref, act_ref, cnt_ref, wg_ref,
                  ylo_ref, yhi_ref, dinv_ref):
    cat = jnp.concatenate([state_ref[...], act_ref[...]], axis=1)
    deg = jnp.sum(cnt_ref[...], axis=0) + 1.0
    dinv = lax.rsqrt(deg)
    xw = jnp.dot(cat, wg_ref[...], preferred_element_type=jnp.float32)
    y = xw * dinv[:, None]
    pad = jnp.zeros((BLK, 5), jnp.float32)
    ylo_ref[...] = jnp.concatenate([y[:, :11], pad], 1).astype(jnp.bfloat16)
    yhi_ref[...] = jnp.concatenate([y[:, 11:22], pad], 1).astype(jnp.bfloat16)
    dinv_ref[...] = dinv[:, None]


def _tc_readout_body(a1_ref, a2_ref, ylo_ref, yhi_ref, dinv_ref,
                     state_ref, act_ref, bg_ref, w1_ref, b1_ref,
                     w2_ref, b2_ref, w3_ref, b3_ref, out_ref):
    f32 = jnp.float32
    acc_lo = (a1_ref[0].astype(f32) + a1_ref[1].astype(f32)
              + ylo_ref[...].astype(f32))
    acc_hi = (a2_ref[0].astype(f32) + a2_ref[1].astype(f32)
              + yhi_ref[...].astype(f32))
    agg = jnp.concatenate([acc_lo[:, :11], acc_hi[:, :11]], axis=1)
    cat = jnp.concatenate([state_ref[...], act_ref[...]], axis=1)
    dinv = dinv_ref[...]
    t = jnp.maximum(dinv * agg + bg_ref[...], 0.0) + cat
    h1 = jnp.maximum(
        jnp.dot(t, w1_ref[...], preferred_element_type=f32)
        + b1_ref[...], 0.0)
    h2 = jnp.maximum(
        jnp.dot(h1, w2_ref[...], preferred_element_type=f32)
        + b2_ref[...], 0.0)
    rows = lax.broadcasted_iota(jnp.int32, (OUT_BLK, BLK), 1)
    grps = lax.broadcasted_iota(jnp.int32, (OUT_BLK, BLK), 0)
    sel = (rows // 6 == grps).astype(f32)
    ssum = jnp.dot(sel, h2, preferred_element_type=f32)
    out = jnp.dot(ssum, w3_ref[...], preferred_element_type=f32)
    out_ref[...] = (out + b3_ref[...]).reshape(OUT_BLK)


def kernel(state, edge_index, action, Wg, bg, W1, b1, W2, b2, W3, b3):
    f32 = jnp.float32
    ei = edge_index.astype(jnp.int32)
    fill = jnp.full((EPAD - N_EDGES,), N_NODES, jnp.int32)
    src2 = jnp.concatenate([ei[0], fill]).reshape(EROWS, 128)
    dst2 = jnp.concatenate([ei[1], fill]).reshape(EROWS, 128)
    state_p = jnp.pad(state, ((0, NPAD - N_NODES), (0, 0)))
    act_p = jnp.pad(action, (0, NPAD - N_NODES)).reshape(NPAD, 1)
    zerostw = jnp.zeros((NPAD, TW), jnp.bfloat16)

    cnt = _sc_count(dst2)

    ylo, yhi, dinv = pl.pallas_call(
        _tc_prep_body,
        grid=(GRID,),
        in_specs=[
            pl.BlockSpec((BLK, 21), lambda i: (i, 0)),
            pl.BlockSpec((BLK, 1), lambda i: (i, 0)),
            pl.BlockSpec((NW, BLK), lambda i: (0, i)),
            pl.BlockSpec((22, 22), lambda i: (0, 0)),
        ],
        out_specs=[
            pl.BlockSpec((BLK, TW), lambda i: (i, 0)),
            pl.BlockSpec((BLK, TW), lambda i: (i, 0)),
            pl.BlockSpec((BLK, 1), lambda i: (i, 0)),
        ],
        out_shape=[
            jax.ShapeDtypeStruct((NPAD, TW), jnp.bfloat16),
            jax.ShapeDtypeStruct((NPAD, TW), jnp.bfloat16),
            jax.ShapeDtypeStruct((NPAD, 1), f32),
        ],
    )(state_p, act_p, cnt, Wg)

    acc1 = _sc_aggregate(src2, dst2, ylo, zerostw)
    acc2 = _sc_aggregate(src2, dst2, yhi, zerostw)

    out1d = pl.pallas_call(
        _tc_readout_body,
        grid=(GRID,),
        in_specs=[
            pl.BlockSpec((2, BLK, TW), lambda i: (0, i, 0)),
            pl.BlockSpec((2, BLK, TW), lambda i: (0, i, 0)),
            pl.BlockSpec((BLK, TW), lambda i: (i, 0)),
            pl.BlockSpec((BLK, TW), lambda i: (i, 0)),
            pl.BlockSpec((BLK, 1), lambda i: (i, 0)),
            pl.BlockSpec((BLK, 21), lambda i: (i, 0)),
            pl.BlockSpec((BLK, 1), lambda i: (i, 0)),
            pl.BlockSpec((22,), lambda i: (0,)),
            pl.BlockSpec((22, HID), lambda i: (0, 0)),
            pl.BlockSpec((HID,), lambda i: (0,)),
            pl.BlockSpec((HID, HID), lambda i: (0, 0)),
            pl.BlockSpec((HID,), lambda i: (0,)),
            pl.BlockSpec((HID, 1), lambda i: (0, 0)),
            pl.BlockSpec((1,), lambda i: (0,)),
        ],
        out_specs=pl.BlockSpec((OUT_BLK,), lambda i: (i,)),
        out_shape=jax.ShapeDtypeStruct((GRID * OUT_BLK,), f32),
    )(acc1, acc2, ylo, yhi, dinv, state_p, act_p,
      bg, W1, b1, W2, b2, W3, b3)

    return out1d[: N_NODES // 6]


# speedup vs baseline: 1.1455x; 1.1455x over previous
import functools

import jax
import jax.numpy as jnp
from jax import lax
from jax.experimental import pallas as pl
from jax.experimental.pallas import tpu as pltpu
from jax.experimental.pallas import tpu_sc as plsc

N_NODES = 99996
N_EDGES = 3199872
HID = 32

NPAD = 100608
EPAD = 3211264
NW = 32
CROWS = 49
CH = CROWS * 128
EROWS = EPAD // 128
ROWS_TILE = EROWS // NW
N_CHUNKS = ROWS_TILE // CROWS
STRIPE = NPAD // 16
BLK = 768
GRID = NPAD // BLK
OUT_BLK = BLK // 6
TW = 16

_mesh = plsc.VectorSubcoreMesh(core_axis_name="c", subcore_axis_name="s")


@functools.partial(
    pl.kernel,
    mesh=_mesh,
    out_type=jax.ShapeDtypeStruct((NW, NPAD), jnp.float32),
    scratch_types=[
        pltpu.VMEM((NPAD,), jnp.float32),
        pltpu.VMEM((16, 128), jnp.int32),
    ],
    compiler_params=pltpu.CompilerParams(
        needs_layout_passes=False, use_tc_tiling_on_sc=False),
)
def _sc_count(dst_hbm, out_hbm, cnt_v, idx_v):
    c = lax.axis_index("c")
    s = lax.axis_index("s")
    w = s * 2 + c

    def zero_body(j, carry):
        cnt_v[pl.ds(j * 16, 16)] = jnp.zeros((16,), jnp.float32)
        return carry

    lax.fori_loop(0, NPAD // 16, zero_body, 0)

    ones = jnp.ones((16,), jnp.float32)

    def chunk_body(g, carry):
        rowbase = w * ROWS_TILE + g * 16
        pltpu.sync_copy(dst_hbm.at[pl.ds(rowbase, 16)], idx_v)

        def inner(t, carry2):
            j = t // 8
            l = t - j * 8
            idx = idx_v[j, pl.ds(l * 16, 16)]
            plsc.addupdate_scatter(cnt_v, [idx], ones)
            return carry2

        lax.fori_loop(0, 128, inner, 0)
        return carry

    lax.fori_loop(0, ROWS_TILE // 16, chunk_body, 0)
    pltpu.sync_copy(cnt_v, out_hbm.at[w])


@functools.partial(
    pl.kernel,
    mesh=_mesh,
    out_type=jax.ShapeDtypeStruct((2, NPAD, TW), jnp.bfloat16),
    scratch_types=[
        pltpu.VMEM((CROWS, 128), jnp.int32),
        pltpu.VMEM((CROWS, 128), jnp.int32),
        pltpu.VMEM((CH, TW), jnp.bfloat16),
        pltpu.VMEM_SHARED((NPAD, TW), jnp.bfloat16),
        pltpu.SemaphoreType.DMA,
        pltpu.SemaphoreType.DMA,
    ],
    compiler_params=pltpu.CompilerParams(
        needs_layout_passes=False, use_tc_tiling_on_sc=False),
)
def _sc_aggregate(src_hbm, dst_hbm, tab_hbm, zeros_hbm, out_hbm,
                  si_v, di_v, rows_v, acc_sh, gsem, ssem):
    c = lax.axis_index("c")
    s = lax.axis_index("s")
    w = s * 2 + c
    row0 = s * STRIPE

    pltpu.sync_copy(zeros_hbm.at[pl.ds(row0, STRIPE)],
                    acc_sh.at[pl.ds(row0, STRIPE)])
    plsc.subcore_barrier()

    def chunk_body(g, carry):
        rowbase = w * ROWS_TILE + g * CROWS
        pltpu.sync_copy(src_hbm.at[pl.ds(rowbase, CROWS)], si_v)
        pltpu.sync_copy(dst_hbm.at[pl.ds(rowbase, CROWS)], di_v)
        gathers = []
        for j in range(CROWS):
            gathers.append(pltpu.async_copy(
                tab_hbm.at[si_v.at[j]],
                rows_v.at[pl.ds(j * 128, 128)],
                gsem,
            ))
        for d in gathers:
            d.wait()
        scatters = []
        for j in range(CROWS):
            scatters.append(pltpu.async_copy(
                rows_v.at[pl.ds(j * 128, 128)],
                acc_sh.at[di_v.at[j]],
                ssem,
                add=True,
            ))
        for d in scatters:
            d.wait()
        return carry

    lax.fori_loop(0, N_CHUNKS, chunk_body, 0)

    plsc.subcore_barrier()
    pltpu.sync_copy(acc_sh.at[pl.ds(row0, STRIPE)],
                    out_hbm.at[c, pl.ds(row0, STRIPE)])


def _tc_prep_body(state_ref, act_ref, cnt_ref, wg_ref,
                  ylo_ref, yhi_ref, dinv_ref):
    cat = jnp.concatenate([state_ref[...], act_ref[...]], axis=1)
    deg = jnp.sum(cnt_ref[...], axis=0) + 1.0
    dinv = lax.rsqrt(deg)
    xw = jnp.dot(cat, wg_ref[...], preferred_element_type=jnp.float32)
    y = xw * dinv[:, None]
    pad = jnp.zeros((BLK, 5), jnp.float32)
    ylo_ref[...] = jnp.concatenate([y[:, :11], pad], 1).astype(jnp.bfloat16)
    yhi_ref[...] = jnp.concatenate([y[:, 11:22], pad], 1).astype(jnp.bfloat16)
    dinv_ref[...] = dinv[:, None]


def _tc_readout_body(a1_ref, a2_ref, ylo_ref, yhi_ref, dinv_ref,
                     state_ref, act_ref, bg_ref, w1_ref, b1_ref,
                     w2_ref, b2_ref, w3_ref, b3_ref, out_ref):
    f32 = jnp.float32
    acc_lo = (a1_ref[0].astype(f32) + a1_ref[1].astype(f32)
              + ylo_ref[...].astype(f32))
    acc_hi = (a2_ref[0].astype(f32) + a2_ref[1].astype(f32)
              + yhi_ref[...].astype(f32))
    agg = jnp.concatenate([acc_lo[:, :11], acc_hi[:, :11]], axis=1)
    cat = jnp.concatenate([state_ref[...], act_ref[...]], axis=1)
    dinv = dinv_ref[...]
    t = jnp.maximum(dinv * agg + bg_ref[...], 0.0) + cat
    h1 = jnp.maximum(
        jnp.dot(t, w1_ref[...], preferred_element_type=f32)
        + b1_ref[...], 0.0)
    h2 = jnp.maximum(
        jnp.dot(h1, w2_ref[...], preferred_element_type=f32)
        + b2_ref[...], 0.0)
    rows = lax.broadcasted_iota(jnp.int32, (OUT_BLK, BLK), 1)
    grps = lax.broadcasted_iota(jnp.int32, (OUT_BLK, BLK), 0)
    sel = (rows // 6 == grps).astype(f32)
    ssum = jnp.dot(sel, h2, preferred_element_type=f32)
    out = jnp.dot(ssum, w3_ref[...], preferred_element_type=f32)
    out_ref[...] = (out + b3_ref[...]).reshape(OUT_BLK)


def kernel(state, edge_index, action, Wg, bg, W1, b1, W2, b2, W3, b3):
    f32 = jnp.float32
    ei = edge_index.astype(jnp.int32)
    fill = jnp.full((EPAD - N_EDGES,), N_NODES, jnp.int32)
    src2 = jnp.concatenate([ei[0], fill]).reshape(EROWS, 128)
    dst2 = jnp.concatenate([ei[1], fill]).reshape(EROWS, 128)
    state_p = jnp.pad(state, ((0, NPAD - N_NODES), (0, 0)))
    act_p = jnp.pad(action, (0, NPAD - N_NODES)).reshape(NPAD, 1)
    zerostw = jnp.zeros((NPAD, TW), jnp.bfloat16)

    cnt = _sc_count(dst2)

    ylo, yhi, dinv = pl.pallas_call(
        _tc_prep_body,
        grid=(GRID,),
        in_specs=[
            pl.BlockSpec((BLK, 21), lambda i: (i, 0)),
            pl.BlockSpec((BLK, 1), lambda i: (i, 0)),
            pl.BlockSpec((NW, BLK), lambda i: (0, i)),
            pl.BlockSpec((22, 22), lambda i: (0, 0)),
        ],
        out_specs=[
            pl.BlockSpec((BLK, TW), lambda i: (i, 0)),
            pl.BlockSpec((BLK, TW), lambda i: (i, 0)),
            pl.BlockSpec((BLK, 1), lambda i: (i, 0)),
        ],
        out_shape=[
            jax.ShapeDtypeStruct((NPAD, TW), jnp.bfloat16),
            jax.ShapeDtypeStruct((NPAD, TW), jnp.bfloat16),
            jax.ShapeDtypeStruct((NPAD, 1), f32),
        ],
    )(state_p, act_p, cnt, Wg)

    acc1 = _sc_aggregate(src2, dst2, ylo, zerostw)
    acc2 = _sc_aggregate(src2, dst2, yhi, zerostw)

    out1d = pl.pallas_call(
        _tc_readout_body,
        grid=(GRID,),
        in_specs=[
            pl.BlockSpec((2, BLK, TW), lambda i: (0, i, 0)),
            pl.BlockSpec((2, BLK, TW), lambda i: (0, i, 0)),
            pl.BlockSpec((BLK, TW), lambda i: (i, 0)),
            pl.BlockSpec((BLK, TW), lambda i: (i, 0)),
            pl.BlockSpec((BLK, 1), lambda i: (i, 0)),
            pl.BlockSpec((BLK, 21), lambda i: (i, 0)),
            pl.BlockSpec((BLK, 1), lambda i: (i, 0)),
            pl.BlockSpec((22,), lambda i: (0,)),
            pl.BlockSpec((22, HID), lambda i: (0, 0)),
            pl.BlockSpec((HID,), lambda i: (0,)),
            pl.BlockSpec((HID, HID), lambda i: (0, 0)),
            pl.BlockSpec((HID,), lambda i: (0,)),
            pl.BlockSpec((HID, 1), lambda i: (0, 0)),
            pl.BlockSpec((1,), lambda i: (0,)),
        ],
        out_specs=pl.BlockSpec((OUT_BLK,), lambda i: (i,)),
        out_shape=jax.ShapeDtypeStruct((GRID * OUT_BLK,), f32),
    )(acc1, acc2, ylo, yhi, dinv, state_p, act_p,
      bg, W1, b1, W2, b2, W3, b3)

    return out1d[: N_NODES // 6]
